# baseline (speedup 1.0000x reference)
import jax
import jax.numpy as jnp
from jax import lax
from jax.experimental import pallas as pl
from jax.experimental.pallas import tpu as pltpu
from jax.experimental.pallas import tpu_sc as plsc

B = 1024
L = 200
D = 768
N = B * L
NC = 2
NS = 16
NW = NC * NS
LANES = 16
ROWS_PER_W = N // NW
W = 64
CHUNKS = ROWS_PER_W // W


def _comb_tc_body(pe_ref, seg_ref, out_ref):
    pe = pe_ref[...]
    seg = seg_ref[...]
    out_ref[...] = (pe[:, None, :] + seg[None, :, :]).reshape(L * 3, D)


def _build_comb(pe_l, seg_w):
    return pl.pallas_call(
        _comb_tc_body,
        out_shape=jax.ShapeDtypeStruct((L * 3, D), jnp.float32),
    )(pe_l, seg_w)


def _sc_body(tok_hbm, comb_hbm, ti_hbm, sl_hbm, out_hbm,
             ti_v, ci_v, tok_buf, comb_buf, sem_t, sem_c):
    wid = lax.axis_index("s") * NC + lax.axis_index("c")
    base = wid * ROWS_PER_W

    @pl.loop(0, CHUNKS)
    def _chunk(j):
        start = base + j * W
        pltpu.sync_copy(ti_hbm.at[pl.ds(start, W)], ti_v)
        pltpu.sync_copy(sl_hbm.at[pl.ds(start, W)], ci_v)

        @pl.loop(0, W // LANES)
        def _ci(v):
            flat = start + v * LANES + lax.iota(jnp.int32, LANES)
            s = ci_v.at[pl.ds(v * LANES, LANES)][...]
            ci_v.at[pl.ds(v * LANES, LANES)][...] = lax.rem(flat, L) * 3 + s

        cp_t = pltpu.async_copy(tok_hbm.at[ti_v], tok_buf, sem_t)
        cp_c = pltpu.async_copy(comb_hbm.at[ci_v], comb_buf, sem_c)
        cp_t.wait()
        cp_c.wait()

        @pl.loop(0, W)
        def _row(r):
            @pl.loop(0, D, step=LANES)
            def _col(c):
                tok_buf.at[r, pl.ds(c, LANES)][...] = (
                    tok_buf.at[r, pl.ds(c, LANES)][...]
                    + comb_buf.at[r, pl.ds(c, LANES)][...])

        pltpu.sync_copy(tok_buf, out_hbm.at[pl.ds(start, W)])


def kernel(x, segment_label, token_weight, segment_weight, pe):
    ti = x.reshape(N).astype(jnp.int32)
    sl = segment_label.reshape(N).astype(jnp.int32)
    comb = _build_comb(pe[0, :L], segment_weight)

    mesh = plsc.VectorSubcoreMesh(core_axis_name="c", subcore_axis_name="s")
    sc = pl.kernel(
        _sc_body,
        out_type=jax.ShapeDtypeStruct((N, D), jnp.float32),
        mesh=mesh,
        scratch_types=[
            pltpu.VMEM((W,), jnp.int32),
            pltpu.VMEM((W,), jnp.int32),
            pltpu.VMEM((W, D), jnp.float32),
            pltpu.VMEM((W, D), jnp.float32),
            pltpu.SemaphoreType.DMA,
            pltpu.SemaphoreType.DMA,
        ],
    )
    out = sc(token_weight, comb, ti, sl)
    return out.reshape(B, L, D)


# speedup vs baseline: 2.5735x; 2.5735x over previous
import jax
import jax.numpy as jnp
from jax import lax
from jax.experimental import pallas as pl
from jax.experimental.pallas import tpu as pltpu
from jax.experimental.pallas import tpu_sc as plsc

B = 1024
L = 200
D = 768
N = B * L
NC = 2
NS = 16
NW = NC * NS
LANES = 16
ROWS_PER_W = N // NW
W = 32
CHUNKS = ROWS_PER_W // W


def _comb_tc_body(pe_ref, seg_ref, out_ref):
    pe = pe_ref[...]
    seg = seg_ref[...]
    out_ref[...] = (pe[:, None, :] + seg[None, :, :]).reshape(L * 3, D)


def _build_comb(pe_l, seg_w):
    return pl.pallas_call(
        _comb_tc_body,
        out_shape=jax.ShapeDtypeStruct((L * 3, D), jnp.float32),
    )(pe_l, seg_w)


def _sc_body(tok_hbm, comb_hbm, ti_hbm, sl_hbm, out_hbm,
             ti0, ci0, tok0, comb0, ti1, ci1, tok1, comb1,
             sem_t0, sem_c0, sem_w0, sem_t1, sem_c1, sem_w1):
    wid = lax.axis_index("s") * NC + lax.axis_index("c")
    base = wid * ROWS_PER_W

    sets = (
        (ti0, ci0, tok0, comb0, sem_t0, sem_c0, sem_w0),
        (ti1, ci1, tok1, comb1, sem_t1, sem_c1, sem_w1),
    )

    def issue(start, p):
        ti_v, ci_v, tok_v, comb_v, sem_t, sem_c, _ = sets[p]
        pltpu.sync_copy(ti_hbm.at[pl.ds(start, W)], ti_v)
        pltpu.sync_copy(sl_hbm.at[pl.ds(start, W)], ci_v)

        @pl.loop(0, W // LANES)
        def _ci(v):
            flat = start + v * LANES + lax.iota(jnp.int32, LANES)
            s = ci_v.at[pl.ds(v * LANES, LANES)][...]
            ci_v.at[pl.ds(v * LANES, LANES)][...] = lax.rem(flat, L) * 3 + s

        pltpu.async_copy(tok_hbm.at[ti_v], tok_v, sem_t)
        pltpu.async_copy(comb_hbm.at[ci_v], comb_v, sem_c)

    def wait_gathers(p):
        ti_v, ci_v, tok_v, comb_v, sem_t, sem_c, _ = sets[p]
        pltpu.make_async_copy(tok_hbm.at[ti_v], tok_v, sem_t).wait()
        pltpu.make_async_copy(comb_hbm.at[ci_v], comb_v, sem_c).wait()

    def add(p):
        _, _, tok_v, comb_v, _, _, _ = sets[p]

        @plsc.parallel_loop(0, W, unroll=2)
        def _row(r):
            for c in range(0, D, LANES):
                tok_v.at[r, pl.ds(c, LANES)][...] = (
                    tok_v.at[r, pl.ds(c, LANES)][...]
                    + comb_v.at[r, pl.ds(c, LANES)][...])

    def start_write(start, p):
        _, _, tok_v, _, _, _, sem_w = sets[p]
        pltpu.async_copy(tok_v, out_hbm.at[pl.ds(start, W)], sem_w)

    def wait_write(start, p):
        _, _, tok_v, _, _, _, sem_w = sets[p]
        pltpu.make_async_copy(tok_v, out_hbm.at[pl.ds(start, W)], sem_w).wait()

    issue(base, 0)

    @pl.loop(0, CHUNKS, step=2)
    def _chunk(j):
        s0 = base + j * W
        s1 = s0 + W

        @pl.when(j > 0)
        def _():
            wait_write(s1 - 2 * W, 1)

        issue(s1, 1)
        wait_gathers(0)
        add(0)
        start_write(s0, 0)
        wait_gathers(1)
        add(1)
        wait_write(s0, 0)

        @pl.when(j + 2 < CHUNKS)
        def _():
            issue(s0 + 2 * W, 0)

        start_write(s1, 1)

    wait_write(base + (CHUNKS - 1) * W, 1)


def kernel(x, segment_label, token_weight, segment_weight, pe):
    ti = x.reshape(N).astype(jnp.int32)
    sl = segment_label.reshape(N).astype(jnp.int32)
    comb = _build_comb(pe[0, :L], segment_weight)

    mesh = plsc.VectorSubcoreMesh(core_axis_name="c", subcore_axis_name="s")
    sc = pl.kernel(
        _sc_body,
        out_type=jax.ShapeDtypeStruct((N, D), jnp.float32),
        mesh=mesh,
        scratch_types=[
            pltpu.VMEM((W,), jnp.int32),
            pltpu.VMEM((W,), jnp.int32),
            pltpu.VMEM((W, D), jnp.float32),
            pltpu.VMEM((W, D), jnp.float32),
            pltpu.VMEM((W,), jnp.int32),
            pltpu.VMEM((W,), jnp.int32),
            pltpu.VMEM((W, D), jnp.float32),
            pltpu.VMEM((W, D), jnp.float32),
            pltpu.SemaphoreType.DMA,
            pltpu.SemaphoreType.DMA,
            pltpu.SemaphoreType.DMA,
            pltpu.SemaphoreType.DMA,
            pltpu.SemaphoreType.DMA,
            pltpu.SemaphoreType.DMA,
        ],
    )
    out = sc(token_weight, comb, ti, sl)
    return out.reshape(B, L, D)
